# baseline (speedup 1.0000x reference)
import functools

import jax
import jax.numpy as jnp
from jax import lax
from jax.experimental import pallas as pl
from jax.experimental.pallas import tpu as pltpu
from jax.experimental.pallas import tpu_sc as plsc

BATCH = 4096
SEQ_LEN = 200
D_MODEL = 64

NC = 2
NS = 16
NW = NC * NS

B = BATCH * SEQ_LEN
B_PER_W = B // NW
CHUNK = 128
STEPS = B_PER_W // CHUNK
NBUF = 4
NGROUPS = STEPS // NBUF

_mesh = plsc.VectorSubcoreMesh(core_axis_name="c", subcore_axis_name="s")


@functools.partial(
    pl.kernel,
    mesh=_mesh,
    compiler_params=pltpu.CompilerParams(use_tc_tiling_on_sc=False),
    out_type=jax.ShapeDtypeStruct((B, D_MODEL), jnp.float32),
    scratch_types=[
        pltpu.VMEM((STEPS, CHUNK), jnp.int32),
        pltpu.VMEM((NBUF, CHUNK, D_MODEL), jnp.float32),
        pltpu.SemaphoreType.DMA((NBUF,)),
    ],
)
def _emb_lookup(idx_hbm, table_hbm, out_hbm, idx_v, rows_v, gsem):
    wid = lax.axis_index("s") * NC + lax.axis_index("c")
    base = wid * B_PER_W

    pltpu.sync_copy(idx_hbm.at[wid], idx_v)

    def fire(j, b):
        pltpu.async_copy(table_hbm.at[idx_v.at[j]], rows_v.at[b], gsem.at[b])

    def wait(b):
        pltpu.make_async_copy(
            table_hbm.at[pl.ds(0, CHUNK)], rows_v.at[b], gsem.at[b]
        ).wait()

    def write(j, b):
        pltpu.sync_copy(rows_v.at[b], out_hbm.at[pl.ds(base + j * CHUNK, CHUNK)])

    for b in range(NBUF):
        fire(b, b)

    def group(g, carry):
        for b in range(NBUF):
            j = g * NBUF + b
            wait(b)
            write(j, b)
            fire(j + NBUF, b)
        return carry

    lax.fori_loop(0, NGROUPS - 1, group, 0, unroll=False)

    for b in range(NBUF):
        j = (NGROUPS - 1) * NBUF + b
        wait(b)
        write(j, b)


def kernel(char_tokens, table):
    idx = char_tokens.reshape(NW, STEPS, CHUNK).astype(jnp.int32)
    out = _emb_lookup(idx, table)
    return out.reshape(BATCH, SEQ_LEN, D_MODEL)
